# baseline (speedup 1.0000x reference)
import math

import jax
import jax.numpy as jnp
from jax import lax
from jax.experimental import pallas as pl
from jax.experimental.pallas import tpu as pltpu
from jax.experimental.pallas import tpu_sc as plsc

_NUM_EMB = 1_000_000
_HALF = _NUM_EMB // 2
_DIM = 32
_K = 8
_ROWS = 16384
_COLS = 26
_N = _ROWS * _COLS
_NC = 2
_NS = 16
_NW = _NC * _NS
_PER_W = _N // _NW
_E = 128
_UNITS = _PER_W // _E
_LANES = 16
_NBUF = 2
_SCALE = 1.0 / math.sqrt(_K)


def _body(table_hbm, ids_hbm, out_hbm, ids_all, idx_vs, gbufs, stg, sem_g,
          sem_o):
    wid = lax.axis_index("s") * _NC + lax.axis_index("c")
    wbase = wid * jnp.int32(_PER_W)
    pltpu.sync_copy(ids_hbm.at[pl.ds(wbase, _PER_W)], ids_all)

    def compute_idx(u, b):
        off = u * jnp.int32(_E)

        def grp(i, c):
            i16 = i * jnp.int32(_LANES)
            x = ids_all[pl.ds(off + i16, _LANES)]
            idx_vs[b][pl.ds(i16, _LANES)] = x
            r = x
            for j in range(1, _K):
                r2 = r + r
                r = jnp.where(r >= _HALF, r2 - _NUM_EMB, r2)
                idx_vs[b][pl.ds(jnp.int32(j * _E) + i16, _LANES)] = r
            return c

        lax.fori_loop(jnp.int32(0), jnp.int32(_E // _LANES), grp, jnp.int32(0))

    def reduce_scale(b):
        def s(i, c):
            row = i * jnp.int32(4)
            for rr in range(4):
                for h in (0, _LANES):
                    v = gbufs[b][row + rr, pl.ds(h, _LANES)]
                    for j in range(1, _K):
                        v = v + gbufs[b][jnp.int32(j * _E) + row + rr,
                                         pl.ds(h, _LANES)]
                    stg[b][row + rr, pl.ds(h, _LANES)] = v * _SCALE
            return c

        lax.fori_loop(jnp.int32(0), jnp.int32(_E // 4), s, jnp.int32(0))

    def wait_out(b):
        pltpu.make_async_copy(
            stg[b], out_hbm.at[pl.ds(0, _E)], sem_o[b]).wait()

    def issue(u, b):
        compute_idx(u, b)
        pltpu.async_copy(table_hbm.at[idx_vs[b]], gbufs[b], sem_g[b])

    def complete(u, b, wait_mode):
        pltpu.make_async_copy(
            table_hbm.at[idx_vs[b]], gbufs[b], sem_g[b]).wait()
        if wait_mode == "always":
            wait_out(b)
        elif wait_mode == "guard":
            @pl.when(u >= jnp.int32(_NBUF))
            def _():
                wait_out(b)
        reduce_scale(b)
        pltpu.async_copy(
            stg[b], out_hbm.at[pl.ds(wbase + u * jnp.int32(_E), _E)],
            sem_o[b])

    issue(jnp.int32(0), 0)

    def grp(g, carry):
        u0 = g * jnp.int32(2)
        issue(u0 + 1, 1)
        complete(u0, 0, "guard")
        issue(u0 + 2, 0)
        complete(u0 + 1, 1, "guard")
        return carry

    n_grps = (_UNITS - 2) // 2
    lax.fori_loop(jnp.int32(0), jnp.int32(n_grps), grp, jnp.int32(0))

    issue(jnp.int32(_UNITS - 1), 1)
    complete(jnp.int32(_UNITS - 2), 0, "always")
    complete(jnp.int32(_UNITS - 1), 1, "always")
    for b in range(_NBUF):
        wait_out(b)


def kernel(id_, emb_weight):
    ids = id_.reshape(_N).astype(jnp.int32)
    mesh = plsc.VectorSubcoreMesh(
        core_axis_name="c", subcore_axis_name="s",
        num_cores=_NC, num_subcores=_NS)
    out = pl.kernel(
        _body,
        out_type=jax.ShapeDtypeStruct((_N, _DIM), jnp.float32),
        mesh=mesh,
        compiler_params=pltpu.CompilerParams(use_tc_tiling_on_sc=False),
        scratch_types=[
            pltpu.VMEM((_PER_W,), jnp.int32),
            [pltpu.VMEM((_K * _E,), jnp.int32) for _ in range(_NBUF)],
            [pltpu.VMEM((_K * _E, _DIM), jnp.float32) for _ in range(_NBUF)],
            [pltpu.VMEM((_E, _DIM), jnp.float32) for _ in range(_NBUF)],
            [pltpu.SemaphoreType.DMA for _ in range(_NBUF)],
            [pltpu.SemaphoreType.DMA for _ in range(_NBUF)],
        ],
    )(emb_weight, ids)
    return out.reshape(_ROWS, _COLS, _DIM)


# speedup vs baseline: 1.0339x; 1.0339x over previous
import math

import jax
import jax.numpy as jnp
from jax import lax
from jax.experimental import pallas as pl
from jax.experimental.pallas import tpu as pltpu
from jax.experimental.pallas import tpu_sc as plsc

_NUM_EMB = 1_000_000
_HALF = _NUM_EMB // 2
_DIM = 32
_K = 8
_ROWS = 16384
_COLS = 26
_N = _ROWS * _COLS
_NC = 2
_NS = 16
_NW = _NC * _NS
_PER_W = _N // _NW
_E = 128
_UNITS = _PER_W // _E
_GRPS = _ROWS // _E
_DG = _DIM // 8
_LANES = 16
_NBUF = 4
_SCALE = 1.0 / math.sqrt(_K)


def _body(table_hbm, ids_hbm, out_hbm, ids_all, idx_vs, acc_vs, stg_vs,
          sem_g, sem_o):
    wid = lax.axis_index("s") * _NC + lax.axis_index("c")
    tbase = wid * jnp.int32(_UNITS)
    pltpu.sync_copy(
        ids_hbm.at[pl.ds(tbase * jnp.int32(_E), _PER_W)], ids_all)
    zero = jnp.zeros((_LANES,), jnp.float32)
    lane = lax.iota(jnp.int32, _LANES)

    def compute_idx(u, b):
        off = u * jnp.int32(_E)

        def grp(i, c):
            i16 = i * jnp.int32(_LANES)
            x = ids_all[pl.ds(off + i16, _LANES)]
            idx_vs[b][0][pl.ds(i16, _LANES)] = x
            r = x
            for j in range(1, _K):
                r2 = r + r
                r = jnp.where(r >= _HALF, r2 - _NUM_EMB, r2)
                idx_vs[b][j][pl.ds(i16, _LANES)] = r
            return c

        lax.fori_loop(jnp.int32(0), jnp.int32(_E // _LANES), grp, jnp.int32(0))

    def zero_acc(b):
        def z(i, c):
            row = i * jnp.int32(8)
            for rr in range(8):
                acc_vs[b][row + rr, pl.ds(0, _LANES)] = zero
                acc_vs[b][row + rr, pl.ds(_LANES, _LANES)] = zero
            return c

        lax.fori_loop(jnp.int32(0), jnp.int32(_E // 8), z, jnp.int32(0))

    def transpose_scale(b):
        def t(rg, c):
            ridx = rg * jnp.int32(_LANES) + lane
            for dg in range(_DG):
                for f in range(8):
                    cidx = jnp.full((_LANES,), dg * 8 + f, jnp.int32)
                    v = plsc.load_gather(acc_vs[b], [ridx, cidx])
                    stg_vs[b][dg, f, pl.ds(rg * jnp.int32(_LANES), _LANES)] = (
                        v * _SCALE)
            return c

        lax.fori_loop(jnp.int32(0), jnp.int32(_E // _LANES), t, jnp.int32(0))

    def wait_out(b):
        for dg in range(_DG):
            pltpu.make_async_copy(
                stg_vs[b].at[jnp.int32(dg)],
                out_hbm.at[jnp.int32(0), jnp.int32(dg), jnp.int32(0)],
                sem_o[b]).wait()

    def issue(u, b):
        compute_idx(u, b)
        zero_acc(b)
        for j in range(_K):
            pltpu.async_copy(
                table_hbm.at[idx_vs[b][j]], acc_vs[b], sem_g[b], add=True)

    def complete(u, b, wait_mode):
        for j in range(_K):
            pltpu.make_async_copy(
                table_hbm.at[idx_vs[b][j]], acc_vs[b], sem_g[b]).wait()
        if wait_mode == "always":
            wait_out(b)
        elif wait_mode == "guard":
            @pl.when(u >= jnp.int32(_NBUF))
            def _():
                wait_out(b)
        transpose_scale(b)
        t = tbase + u
        col = t // jnp.int32(_GRPS)
        g = t % jnp.int32(_GRPS)
        for dg in range(_DG):
            pltpu.async_copy(
                stg_vs[b].at[jnp.int32(dg)],
                out_hbm.at[col, jnp.int32(dg), g], sem_o[b])

    for u0 in range(_NBUF - 1):
        issue(jnp.int32(u0), u0)

    def grp(gi, carry):
        for b4 in range(_NBUF):
            u_i = jnp.int32(_NBUF - 1) + gi * jnp.int32(_NBUF) + jnp.int32(b4)
            issue(u_i, (_NBUF - 1 + b4) % _NBUF)
            u_c = gi * jnp.int32(_NBUF) + jnp.int32(b4)
            complete(u_c, b4, "guard")
        return carry

    n_grps = (_UNITS - (_NBUF - 1)) // _NBUF
    lax.fori_loop(jnp.int32(0), jnp.int32(n_grps), grp, jnp.int32(0))

    issue(jnp.int32(_UNITS - 1), (_UNITS - 1) % _NBUF)
    for uc in range(_UNITS - _NBUF, _UNITS):
        complete(jnp.int32(uc), uc % _NBUF, "always")
    for b in range(_NBUF):
        wait_out(b)


def kernel(id_, emb_weight):
    ids = id_.T.reshape(_N).astype(jnp.int32)
    mesh = plsc.VectorSubcoreMesh(
        core_axis_name="c", subcore_axis_name="s",
        num_cores=_NC, num_subcores=_NS)
    out5 = pl.kernel(
        _body,
        out_type=jax.ShapeDtypeStruct((_COLS, _DG, _GRPS, 8, _E),
                                      jnp.float32),
        mesh=mesh,
        compiler_params=pltpu.CompilerParams(
            use_tc_tiling_on_sc=False, needs_layout_passes=False),
        scratch_types=[
            pltpu.VMEM((_PER_W,), jnp.int32),
            [[pltpu.VMEM((_E,), jnp.int32) for _ in range(_K)]
             for _ in range(_NBUF)],
            [pltpu.VMEM((_E, _DIM), jnp.float32) for _ in range(_NBUF)],
            [pltpu.VMEM((_DG, 8, _E), jnp.float32) for _ in range(_NBUF)],
            [pltpu.SemaphoreType.DMA for _ in range(_NBUF)],
            [pltpu.SemaphoreType.DMA for _ in range(_NBUF)],
        ],
    )(emb_weight, ids)
    return out5.transpose(2, 4, 0, 1, 3).reshape(_ROWS, _COLS, _DIM)
